# baseline (speedup 1.0000x reference)
import functools

import jax
import jax.numpy as jnp
from jax import lax
from jax.experimental import pallas as pl
from jax.experimental.pallas import tpu as pltpu
from jax.experimental.pallas import tpu_sc as plsc

N = 10000
E = 160000
D = 256
HALF = 128

NC = 2
NS = 16
CHUNK = 64
CHUNKS = 160
NBUF = 4
NSLOT = 8
EPT = CHUNK * CHUNKS
E_PAD = EPT * NS
ACC_ROWS = 10240
ZROWS = ACC_ROWS // NS
CNT_W = 10240
ROWS_OUT = ACC_ROWS // NS
NGRP = CHUNK // 16


def _sc_body(x2_hbm, ci_hbm, out_hbm, cnt_hbm,
             acc, ci_v, gx_v, rows_v, cnt_v,
             sem_g0, sem_g1, sem_g2, sem_g3,
             sem_s0, sem_s1, sem_s2, sem_s3,
             sem_i0, sem_i1, sem_i2, sem_i3):
    c = lax.axis_index("c")
    s = lax.axis_index("s")
    cbase = s * CHUNKS
    sem_g = (sem_g0, sem_g1, sem_g2, sem_g3)
    sem_s = (sem_s0, sem_s1, sem_s2, sem_s3)
    sem_i = (sem_i0, sem_i1, sem_i2, sem_i3)

    z16 = jnp.zeros((16,), jnp.float32)

    def zrow_body(r, carry):
        for k in range(HALF // 16):
            rows_v[0, r, pl.ds(k * 16, 16)] = z16
        return carry

    lax.fori_loop(0, CHUNK, zrow_body, 0)

    def zcnt_body(t, carry):
        cnt_v[pl.ds(t * 16, 16)] = z16
        return carry

    lax.fori_loop(0, CNT_W // 16, zcnt_body, 0)

    for i in range(ZROWS // CHUNK):
        pltpu.sync_copy(rows_v.at[0],
                        acc.at[pl.ds(s * ZROWS + i * CHUNK, CHUNK)])

    plsc.subcore_barrier()

    ones = jnp.ones((16,), jnp.float32)

    def fill_gx(slot, buf):
        for k in range(NGRP):
            v = ci_v[slot, 0, pl.ds(k * 16, 16)]
            gx_v[buf, pl.ds(k * 16, 16)] = v * 2 + c

    def start_gather(buf, sem):
        pltpu.async_copy(x2_hbm.at[gx_v.at[buf]], rows_v.at[buf], sem)

    pltpu.sync_copy(ci_hbm.at[pl.ds(cbase, 2)], ci_v.at[pl.ds(0, 2)])
    for sl in range(2, 6):
        pltpu.async_copy(ci_hbm.at[pl.ds(cbase + sl, 1)],
                         ci_v.at[pl.ds(sl, 1)], sem_i[sl % 4])
    fill_gx(0, 0)
    start_gather(0, sem_g[0])
    fill_gx(1, 1)
    start_gather(1, sem_g[1])

    def chunk_body(j, carry):
        q = j % NSLOT
        q2 = (j + 2) % NSLOT
        q6 = (j + 6) % NSLOT

        for k in range(NGRP):
            cidx = ci_v[q, 1, pl.ds(k * 16, 16)]
            plsc.addupdate_scatter(cnt_v, [cidx], ones)

        def on_parity(b_):
            b2 = (b_ + 2) % NBUF
            pltpu.make_async_copy(x2_hbm.at[gx_v.at[b_]], rows_v.at[b_],
                                  sem_g[b_]).wait()
            pltpu.async_copy(rows_v.at[b_], acc.at[ci_v.at[q, 1]],
                             sem_s[b_], add=True)

            @pl.when(j + 2 < CHUNKS)
            def _():
                pltpu.make_async_copy(
                    ci_hbm.at[pl.ds(cbase + j + 2, 1)],
                    ci_v.at[pl.ds(q2, 1)], sem_i[(b_ + 2) % 4]).wait()

                @pl.when(j >= 2)
                def _():
                    pltpu.make_async_copy(
                        rows_v.at[b2], acc.at[ci_v.at[q2, 1]],
                        sem_s[b2]).wait()

                fill_gx(q2, b2)
                start_gather(b2, sem_g[b2])

            @pl.when(j + 6 < CHUNKS)
            def _():
                pltpu.async_copy(ci_hbm.at[pl.ds(cbase + j + 6, 1)],
                                 ci_v.at[pl.ds(q6, 1)], sem_i[(b_ + 2) % 4])

        for b_ in range(NBUF):
            @pl.when(j % NBUF == b_)
            def _(b_=b_):
                on_parity(b_)

        return carry

    lax.fori_loop(0, CHUNKS, chunk_body, 0)

    for jj in range(CHUNKS - 4, CHUNKS):
        pltpu.make_async_copy(rows_v.at[jj % NBUF],
                              acc.at[ci_v.at[jj % NSLOT, 1]],
                              sem_s[jj % NBUF]).wait()

    @pl.when(c == 0)
    def _():
        pltpu.sync_copy(cnt_v, cnt_hbm.at[s])

    plsc.subcore_barrier()

    rbase = s * ROWS_OUT
    pltpu.sync_copy(acc.at[pl.ds(rbase, ROWS_OUT)],
                    out_hbm.at[pl.ds(c * ACC_ROWS + rbase, ROWS_OUT)])


_sc_aggregate = functools.partial(
    pl.kernel,
    out_type=(
        jax.ShapeDtypeStruct((2 * ACC_ROWS, HALF), jnp.float32),
        jax.ShapeDtypeStruct((NS, CNT_W), jnp.float32),
    ),
    mesh=plsc.VectorSubcoreMesh(core_axis_name="c", subcore_axis_name="s"),
    compiler_params=pltpu.CompilerParams(needs_layout_passes=False),
    scratch_types=[
        pltpu.VMEM_SHARED((ACC_ROWS, HALF), jnp.float32),
        pltpu.VMEM((NSLOT, 2, CHUNK), jnp.int32),
        pltpu.VMEM((NBUF, CHUNK), jnp.int32),
        pltpu.VMEM((NBUF, CHUNK, HALF), jnp.float32),
        pltpu.VMEM((CNT_W,), jnp.float32),
        pltpu.SemaphoreType.DMA,
        pltpu.SemaphoreType.DMA,
        pltpu.SemaphoreType.DMA,
        pltpu.SemaphoreType.DMA,
        pltpu.SemaphoreType.DMA,
        pltpu.SemaphoreType.DMA,
        pltpu.SemaphoreType.DMA,
        pltpu.SemaphoreType.DMA,
        pltpu.SemaphoreType.DMA,
        pltpu.SemaphoreType.DMA,
        pltpu.SemaphoreType.DMA,
        pltpu.SemaphoreType.DMA,
    ],
)(_sc_body)


def _epi_body(cnt_ref, s0_ref, s1_ref, x_ref, wl_ref, bl_ref, wr_ref,
              wo_ref, bo_ref, o_ref):
    dn = (((1,), (1,)), ((), ()))
    cnt = jnp.sum(cnt_ref[...], axis=0)
    inv = 1.0 / jnp.maximum(cnt, 1.0)
    m0 = (s0_ref[...] * inv[:, None]).astype(jnp.bfloat16)
    m1 = (s1_ref[...] * inv[:, None]).astype(jnp.bfloat16)
    wl = wl_ref[...].astype(jnp.bfloat16)
    h = lax.dot_general(m0, wl[:, :HALF], dn, preferred_element_type=jnp.float32)
    h = h + lax.dot_general(m1, wl[:, HALF:], dn, preferred_element_type=jnp.float32)
    h = h + lax.dot_general(x_ref[...].astype(jnp.bfloat16),
                            wr_ref[...].astype(jnp.bfloat16), dn,
                            preferred_element_type=jnp.float32)
    h = h + bl_ref[...]
    h = jnp.where(h > 0, h, jnp.exp(h) - 1.0)
    o = lax.dot_general(h.astype(jnp.bfloat16),
                        wo_ref[...].astype(jnp.bfloat16), dn,
                        preferred_element_type=jnp.float32)
    o_ref[...] = o + bo_ref[...]


BLK = 512
GRID = (N + BLK - 1) // BLK


def _epilogue(cnt16, summed2, x, W_l, b_l, W_r, W_out, b_out):
    return pl.pallas_call(
        _epi_body,
        grid=(GRID,),
        in_specs=[
            pl.BlockSpec((NS, BLK), lambda i: (0, i)),
            pl.BlockSpec((BLK, HALF), lambda i: (i, 0)),
            pl.BlockSpec((BLK, HALF), lambda i: (i + GRID, 0)),
            pl.BlockSpec((BLK, D), lambda i: (i, 0)),
            pl.BlockSpec((D, D), lambda i: (0, 0)),
            pl.BlockSpec((1, D), lambda i: (0, 0)),
            pl.BlockSpec((D, D), lambda i: (0, 0)),
            pl.BlockSpec((D, D), lambda i: (0, 0)),
            pl.BlockSpec((1, D), lambda i: (0, 0)),
        ],
        out_specs=pl.BlockSpec((BLK, D), lambda i: (i, 0)),
        out_shape=jax.ShapeDtypeStruct((N, D), jnp.float32),
    )(cnt16, summed2, summed2, x, W_l, b_l, W_r, W_out, b_out)


@jax.jit
def kernel(x, edge_index, W_l, b_l, W_r, W_out, b_out):
    src = edge_index[0]
    dst = edge_index[1]
    pad = E_PAD - E
    src_p = jnp.concatenate([src, jnp.arange(pad, dtype=jnp.int32)])
    dst_p = jnp.concatenate([dst, jnp.full((pad,), N, jnp.int32)])
    ci = jnp.stack([src_p.reshape(-1, CHUNK), dst_p.reshape(-1, CHUNK)],
                   axis=1)
    x2 = x.reshape(2 * N, HALF)

    summed2, cnt16 = _sc_aggregate(x2, ci)
    return _epilogue(cnt16, summed2, x, W_l,
                     b_l.reshape(1, D), W_r, W_out, b_out.reshape(1, D))


# speedup vs baseline: 1.1020x; 1.1020x over previous
import functools

import jax
import jax.numpy as jnp
from jax import lax
from jax.experimental import pallas as pl
from jax.experimental.pallas import tpu as pltpu
from jax.experimental.pallas import tpu_sc as plsc

N = 10000
E = 160000
D = 256
HALF = 128

NC = 2
NS = 16
CHUNK = 128
CHUNKS = 80
EPT = CHUNK * CHUNKS
E_PAD = EPT * NS
ACC_ROWS = 10240
ZROWS = ACC_ROWS // NS
CNT_W = 10240
ROWS_OUT = ACC_ROWS // NS
NGRP = CHUNK // 16


def _sc_body(x2_hbm, ci_hbm, out_hbm, cnt_hbm,
             acc, ci_v, gx_v, rows_v, cnt_v,
             sem_g0, sem_g1, sem_i0, sem_i1):
    c = lax.axis_index("c")
    s = lax.axis_index("s")
    cbase = s * CHUNKS

    z16 = jnp.zeros((16,), jnp.float32)

    def zrow_body(r, carry):
        for bi in range(2):
            for k in range(NGRP):
                rows_v[bi, r, pl.ds(k * 16, 16)] = z16
        return carry

    lax.fori_loop(0, CHUNK, zrow_body, 0)

    def zcnt_body(t, carry):
        cnt_v[pl.ds(t * 16, 16)] = z16
        return carry

    lax.fori_loop(0, CNT_W // 16, zcnt_body, 0)

    for i in range(ZROWS // CHUNK):
        pltpu.sync_copy(rows_v.at[0],
                        acc.at[pl.ds(s * ZROWS + i * CHUNK, CHUNK)])

    plsc.subcore_barrier()

    ones = jnp.ones((16,), jnp.float32)

    def fill_gx(slot, buf):
        for k in range(NGRP):
            v = ci_v[slot, 0, pl.ds(k * 16, 16)]
            gx_v[buf, pl.ds(k * 16, 16)] = v * 2 + c

    def start_gather(buf, sem):
        pltpu.async_copy(x2_hbm.at[gx_v.at[buf]], rows_v.at[buf], sem)

    pltpu.sync_copy(ci_hbm.at[pl.ds(cbase, 2)], ci_v.at[pl.ds(0, 2)])
    pltpu.async_copy(ci_hbm.at[pl.ds(cbase + 2, 1)], ci_v.at[pl.ds(2, 1)],
                     sem_i0)
    pltpu.async_copy(ci_hbm.at[pl.ds(cbase + 3, 1)], ci_v.at[pl.ds(3, 1)],
                     sem_i1)
    fill_gx(0, 0)
    start_gather(0, sem_g0)
    fill_gx(1, 1)
    start_gather(1, sem_g1)

    def chunk_body(j, carry):
        b = j % 2
        q = j % 4
        q2 = (j + 2) % 4

        def on_parity(b_, sg, si):
            pltpu.make_async_copy(x2_hbm.at[gx_v.at[b_]], rows_v.at[b_],
                                  sg).wait()
            pltpu.sync_copy(rows_v.at[b_], acc.at[ci_v.at[q, 1]], add=True)

            @pl.when(j + 2 < CHUNKS)
            def _():
                pltpu.make_async_copy(
                    ci_hbm.at[pl.ds(cbase + j + 2, 1)],
                    ci_v.at[pl.ds(q2, 1)], si).wait()
                fill_gx(q2, b_)
                start_gather(b_, sg)

            @pl.when(j + 4 < CHUNKS)
            def _():
                pltpu.async_copy(ci_hbm.at[pl.ds(cbase + j + 4, 1)],
                                 ci_v.at[pl.ds(q, 1)], si)

        for k in range(NGRP):
            cidx = ci_v[q, 1, pl.ds(k * 16, 16)]
            plsc.addupdate_scatter(cnt_v, [cidx], ones)

        @pl.when(b == 0)
        def _():
            on_parity(0, sem_g0, sem_i0)

        @pl.when(b == 1)
        def _():
            on_parity(1, sem_g1, sem_i1)

        return carry

    lax.fori_loop(0, CHUNKS, chunk_body, 0)

    @pl.when(c == 0)
    def _():
        pltpu.sync_copy(cnt_v, cnt_hbm.at[s])

    plsc.subcore_barrier()

    rbase = s * ROWS_OUT
    pltpu.sync_copy(acc.at[pl.ds(rbase, ROWS_OUT)],
                    out_hbm.at[pl.ds(c * ACC_ROWS + rbase, ROWS_OUT)])


_sc_aggregate = functools.partial(
    pl.kernel,
    out_type=(
        jax.ShapeDtypeStruct((2 * ACC_ROWS, HALF), jnp.float32),
        jax.ShapeDtypeStruct((NS, CNT_W), jnp.float32),
    ),
    mesh=plsc.VectorSubcoreMesh(core_axis_name="c", subcore_axis_name="s"),
    compiler_params=pltpu.CompilerParams(needs_layout_passes=False),
    scratch_types=[
        pltpu.VMEM_SHARED((ACC_ROWS, HALF), jnp.float32),
        pltpu.VMEM((4, 2, CHUNK), jnp.int32),
        pltpu.VMEM((2, CHUNK), jnp.int32),
        pltpu.VMEM((2, CHUNK, HALF), jnp.float32),
        pltpu.VMEM((CNT_W,), jnp.float32),
        pltpu.SemaphoreType.DMA,
        pltpu.SemaphoreType.DMA,
        pltpu.SemaphoreType.DMA,
        pltpu.SemaphoreType.DMA,
    ],
)(_sc_body)


def _epi_body(cnt_ref, s0_ref, s1_ref, x_ref, wl_ref, bl_ref, wr_ref,
              wo_ref, bo_ref, o_ref):
    dn = (((1,), (1,)), ((), ()))
    cnt = jnp.sum(cnt_ref[...], axis=0)
    inv = 1.0 / jnp.maximum(cnt, 1.0)
    m0 = (s0_ref[...] * inv[:, None]).astype(jnp.bfloat16)
    m1 = (s1_ref[...] * inv[:, None]).astype(jnp.bfloat16)
    wl = wl_ref[...].astype(jnp.bfloat16)
    h = lax.dot_general(m0, wl[:, :HALF], dn, preferred_element_type=jnp.float32)
    h = h + lax.dot_general(m1, wl[:, HALF:], dn, preferred_element_type=jnp.float32)
    h = h + lax.dot_general(x_ref[...].astype(jnp.bfloat16),
                            wr_ref[...].astype(jnp.bfloat16), dn,
                            preferred_element_type=jnp.float32)
    h = h + bl_ref[...]
    h = jnp.where(h > 0, h, jnp.exp(h) - 1.0)
    o = lax.dot_general(h.astype(jnp.bfloat16),
                        wo_ref[...].astype(jnp.bfloat16), dn,
                        preferred_element_type=jnp.float32)
    o_ref[...] = o + bo_ref[...]


BLK = 512
GRID = (N + BLK - 1) // BLK


def _epilogue(cnt16, summed2, x, W_l, b_l, W_r, W_out, b_out):
    return pl.pallas_call(
        _epi_body,
        grid=(GRID,),
        in_specs=[
            pl.BlockSpec((NS, BLK), lambda i: (0, i)),
            pl.BlockSpec((BLK, HALF), lambda i: (i, 0)),
            pl.BlockSpec((BLK, HALF), lambda i: (i + GRID, 0)),
            pl.BlockSpec((BLK, D), lambda i: (i, 0)),
            pl.BlockSpec((D, D), lambda i: (0, 0)),
            pl.BlockSpec((1, D), lambda i: (0, 0)),
            pl.BlockSpec((D, D), lambda i: (0, 0)),
            pl.BlockSpec((D, D), lambda i: (0, 0)),
            pl.BlockSpec((1, D), lambda i: (0, 0)),
        ],
        out_specs=pl.BlockSpec((BLK, D), lambda i: (i, 0)),
        out_shape=jax.ShapeDtypeStruct((N, D), jnp.float32),
    )(cnt16, summed2, summed2, x, W_l, b_l, W_r, W_out, b_out)


@jax.jit
def kernel(x, edge_index, W_l, b_l, W_r, W_out, b_out):
    src = edge_index[0]
    dst = edge_index[1]
    pad = E_PAD - E
    src_p = jnp.concatenate([src, jnp.arange(pad, dtype=jnp.int32)])
    dst_p = jnp.concatenate([dst, jnp.full((pad,), N, jnp.int32)])
    ci = jnp.stack([src_p.reshape(-1, CHUNK), dst_p.reshape(-1, CHUNK)],
                   axis=1)
    x2 = x.reshape(2 * N, HALF)

    summed2, cnt16 = _sc_aggregate(x2, ci)
    return _epilogue(cnt16, summed2, x, W_l,
                     b_l.reshape(1, D), W_r, W_out, b_out.reshape(1, D))
